# Initial kernel scaffold; baseline (speedup 1.0000x reference)
import jax
import jax.numpy as jnp
from jax.experimental import pallas as pl


def kernel(x, edge_index, edge_weight, W1, b1, W2, b2, We, be):
    raise NotImplementedError("write your pallas kernel here")


# speedup vs baseline: 3.2919x; 3.2919x over previous
import functools

import jax
import jax.numpy as jnp
from jax import lax
from jax.experimental import pallas as pl
from jax.experimental.pallas import tpu as pltpu
from jax.experimental.pallas import tpu_sc as plsc

N = 10000
E = 160000
F_IN = 256
HID = 256
H1 = 128

NC = 2
NS = 16
EPT = E // NS
CH = 80
NCHUNK = EPT // CH
STRIPE = 640
STRIPE_LAST = N - 15 * STRIPE
HF = HID // 2

_mesh = plsc.VectorSubcoreMesh(core_axis_name="c", subcore_axis_name="s")
_sc_params = pltpu.CompilerParams(use_tc_tiling_on_sc=False,
                                  needs_layout_passes=False)


def _deg_body(ei3_hbm, deg_hbm, ones_v, zero_v, idx_v, spmem):
    c = lax.axis_index("c")
    s = lax.axis_index("s")

    def fill_ones(i, _):
        ones_v[i] = jnp.ones((16,), jnp.float32)
        return 0

    lax.fori_loop(0, CH, fill_ones, 0)

    def fill_zero(i, _):
        zero_v[i] = jnp.zeros((16,), jnp.float32)
        return 0

    lax.fori_loop(0, STRIPE, fill_zero, 0)

    pltpu.sync_copy(ei3_hbm.at[c, s], idx_v)

    @pl.when(s < NS - 1)
    def _():
        pltpu.sync_copy(zero_v, spmem.at[pl.ds(s * STRIPE, STRIPE)])

    @pl.when(s == NS - 1)
    def _():
        pltpu.sync_copy(zero_v.at[pl.ds(0, STRIPE_LAST)],
                        spmem.at[pl.ds(s * STRIPE, STRIPE_LAST)])

    plsc.subcore_barrier()

    def chunk(j, _):
        pltpu.sync_copy(ones_v, spmem.at[idx_v.at[j]], add=True)
        return 0

    lax.fori_loop(0, NCHUNK, chunk, 0)
    plsc.subcore_barrier()

    @pl.when(s < NS - 1)
    def _():
        pltpu.sync_copy(spmem.at[pl.ds(s * STRIPE, STRIPE)],
                        deg_hbm.at[c, pl.ds(s * STRIPE, STRIPE)])

    @pl.when(s == NS - 1)
    def _():
        pltpu.sync_copy(spmem.at[pl.ds(s * STRIPE, STRIPE_LAST)],
                        deg_hbm.at[c, pl.ds(s * STRIPE, STRIPE_LAST)])


_deg_kernel = pl.kernel(
    _deg_body,
    out_type=jax.ShapeDtypeStruct((2, N, 16), jnp.float32),
    mesh=_mesh,
    compiler_params=_sc_params,
    scratch_types=[
        pltpu.VMEM((CH, 16), jnp.float32),
        pltpu.VMEM((STRIPE, 16), jnp.float32),
        pltpu.VMEM((NCHUNK, CH), jnp.int32),
        pltpu.VMEM_SHARED((N, 16), jnp.float32),
    ],
)


def _msg_body(hs_hbm, src_hbm, ei3_hbm, ew_hbm, agg_hbm,
              src_v, dst_v, ew_v, rows_v, spmem, sem):
    c = lax.axis_index("c")
    s = lax.axis_index("s")
    base = s * EPT

    pltpu.sync_copy(src_hbm.at[pl.ds(base, EPT)], src_v)
    pltpu.sync_copy(ei3_hbm.at[1, s], dst_v)
    pltpu.sync_copy(ew_hbm.at[pl.ds(base, EPT)], ew_v)

    def zr(i, _):
        for k in range(HF // 16):
            rows_v[i, pl.ds(k * 16, 16)] = jnp.zeros((16,), jnp.float32)
        return 0

    lax.fori_loop(0, 80, zr, 0)
    nz = lax.select(s < NS - 1, STRIPE // 80, STRIPE_LAST // 80)

    def zcopy(k, _):
        pltpu.sync_copy(rows_v.at[pl.ds(0, 80)],
                        spmem.at[pl.ds(s * STRIPE + k * 80, 80)])
        return 0

    lax.fori_loop(0, nz, zcopy, 0)
    plsc.subcore_barrier()

    def chunk(j, _):
        idx = src_v.at[pl.ds(j * CH, CH)]
        pltpu.async_copy(hs_hbm.at[c].at[idx], rows_v, sem).wait()

        def row(r, _):
            f = plsc.load_gather(
                ew_v, [jnp.broadcast_to(j * CH + r, (16,)).astype(jnp.int32)])
            for k in range(HF // 16):
                sl = pl.ds(k * 16, 16)
                rows_v[r, sl] = rows_v[r, sl] * f
            return 0

        lax.fori_loop(0, CH, row, 0)
        pltpu.sync_copy(rows_v, spmem.at[dst_v.at[j]], add=True)
        return 0

    lax.fori_loop(0, NCHUNK, chunk, 0)
    plsc.subcore_barrier()

    @pl.when(s < NS - 1)
    def _():
        pltpu.sync_copy(spmem.at[pl.ds(s * STRIPE, STRIPE)],
                        agg_hbm.at[c, pl.ds(s * STRIPE, STRIPE)])

    @pl.when(s == NS - 1)
    def _():
        pltpu.sync_copy(spmem.at[pl.ds(s * STRIPE, STRIPE_LAST)],
                        agg_hbm.at[c, pl.ds(s * STRIPE, STRIPE_LAST)])


_msg_kernel = pl.kernel(
    _msg_body,
    out_type=jax.ShapeDtypeStruct((2, N, HF), jnp.float32),
    mesh=_mesh,
    compiler_params=_sc_params,
    scratch_types=[
        pltpu.VMEM((EPT,), jnp.int32),
        pltpu.VMEM((NCHUNK, CH), jnp.int32),
        pltpu.VMEM((EPT,), jnp.float32),
        pltpu.VMEM((CH, HF), jnp.float32),
        pltpu.VMEM_SHARED((N, HF), jnp.float32),
        pltpu.SemaphoreType.DMA,
    ],
)


def _norm_body(deg_ref, out_ref):
    d = deg_ref[0, :, :1]
    out_ref[0] = lax.rsqrt(jnp.maximum(d, 1.0))


def _norm_kernel(deg16):
    bn = 2000
    return pl.pallas_call(
        _norm_body,
        grid=(2, N // bn),
        in_specs=[pl.BlockSpec((1, bn, 16), lambda a, i: (a, i, 0))],
        out_specs=pl.BlockSpec((1, bn, 1), lambda a, i: (a, i, 0)),
        out_shape=jax.ShapeDtypeStruct((2, N, 1), jnp.float32),
    )(deg16)


def _mmA_body(x_ref, onorm_ref, w_ref, o_ref):
    o_ref[0] = jnp.dot(x_ref[...] * onorm_ref[...], w_ref[...],
                       preferred_element_type=jnp.float32)


def _mmA(x, onorm, W1):
    br = 1000
    return pl.pallas_call(
        _mmA_body,
        grid=(N // br, 2),
        in_specs=[
            pl.BlockSpec((br, F_IN), lambda i, j: (i, 0)),
            pl.BlockSpec((br, 1), lambda i, j: (i, 0)),
            pl.BlockSpec((F_IN, HF), lambda i, j: (0, j)),
        ],
        out_specs=pl.BlockSpec((1, br, HF), lambda i, j: (j, i, 0)),
        out_shape=jax.ShapeDtypeStruct((2, N, HF), jnp.float32),
    )(x, onorm, W1)


def _mmB_body(alo_ref, ahi_ref, inorm_ref, onorm_ref, b_ref, w_ref, o_ref):
    a = jnp.concatenate([alo_ref[0], ahi_ref[0]], axis=1)
    h = jnp.maximum(a * inorm_ref[...] + b_ref[...], 0.0) * onorm_ref[...]
    o_ref[0] = jnp.dot(h, w_ref[...], preferred_element_type=jnp.float32)


def _mmB(agg, inorm, onorm, b, W):
    br = 1000
    return pl.pallas_call(
        _mmB_body,
        grid=(N // br, 2),
        in_specs=[
            pl.BlockSpec((1, br, HF), lambda i, j: (0, i, 0)),
            pl.BlockSpec((1, br, HF), lambda i, j: (1, i, 0)),
            pl.BlockSpec((br, 1), lambda i, j: (i, 0)),
            pl.BlockSpec((br, 1), lambda i, j: (i, 0)),
            pl.BlockSpec((1, HID), lambda i, j: (0, 0)),
            pl.BlockSpec((HID, HF), lambda i, j: (0, j)),
        ],
        out_specs=pl.BlockSpec((1, br, HF), lambda i, j: (j, i, 0)),
        out_shape=jax.ShapeDtypeStruct((2, N, HF), jnp.float32),
    )(agg, agg, inorm, onorm, b, W)


def _mmC1_body(alo_ref, ahi_ref, inorm_ref, b_ref, we_ref, be_ref, o_ref):
    a = jnp.concatenate([alo_ref[0], ahi_ref[0]], axis=1)
    h = jnp.maximum(a * inorm_ref[...] + b_ref[...], 0.0)
    z = jnp.dot(h, we_ref[...], preferred_element_type=jnp.float32)
    o_ref[...] = jnp.maximum(z + be_ref[...], 0.0)


def _mmC1(agg, inorm, b, We, be):
    br = 1000
    return pl.pallas_call(
        _mmC1_body,
        grid=(N // br,),
        in_specs=[
            pl.BlockSpec((1, br, HF), lambda i: (0, i, 0)),
            pl.BlockSpec((1, br, HF), lambda i: (1, i, 0)),
            pl.BlockSpec((br, 1), lambda i: (i, 0)),
            pl.BlockSpec((1, HID), lambda i: (0, 0)),
            pl.BlockSpec((HID, H1), lambda i: (0, 0)),
            pl.BlockSpec((1, H1), lambda i: (0, 0)),
        ],
        out_specs=pl.BlockSpec((br, H1), lambda i: (i, 0)),
        out_shape=jax.ShapeDtypeStruct((N, H1), jnp.float32),
    )(agg, agg, inorm, b, We, be)


def _mmC2_body(zi_ref, zj_ref, o_ref):
    o_ref[...] = lax.dot_general(
        zi_ref[...], zj_ref[...], (((1,), (1,)), ((), ())),
        preferred_element_type=jnp.float32)


def _mmC2(z):
    bi = 400
    return pl.pallas_call(
        _mmC2_body,
        grid=(N // bi,),
        in_specs=[
            pl.BlockSpec((bi, H1), lambda i: (i, 0)),
            pl.BlockSpec((N, H1), lambda i: (0, 0)),
        ],
        out_specs=pl.BlockSpec((bi, N), lambda i: (i, 0)),
        out_shape=jax.ShapeDtypeStruct((N, N), jnp.float32),
    )(z, z)


def kernel(x, edge_index, edge_weight, W1, b1, W2, b2, We, be):
    ei3 = edge_index.reshape(2, NS, NCHUNK, CH)
    b1r = b1.reshape(1, HID)
    b2r = b2.reshape(1, HID)
    ber = be.reshape(1, H1)

    src_flat = edge_index[0]

    deg16 = _deg_kernel(ei3)
    norms = _norm_kernel(deg16)
    onorm = norms[0]
    inorm = norms[1]

    hs1 = _mmA(x, onorm, W1)
    agg1 = _msg_kernel(hs1, src_flat, ei3, edge_weight)
    hs2 = _mmB(agg1, inorm, onorm, b1r, W2)
    agg2 = _msg_kernel(hs2, src_flat, ei3, edge_weight)
    z = _mmC1(agg2, inorm, b2r, We, ber)
    adj = _mmC2(z)
    return (adj, z)


# speedup vs baseline: 4.8693x; 1.4792x over previous
import functools

import jax
import jax.numpy as jnp
from jax import lax
from jax.experimental import pallas as pl
from jax.experimental.pallas import tpu as pltpu
from jax.experimental.pallas import tpu_sc as plsc

N = 10000
E = 160000
F_IN = 256
HID = 256
H1 = 128

NC = 2
NS = 16
EPT = E // NS
CH = 80
NCHUNK = EPT // CH
STRIPE = 640
STRIPE_LAST = N - 15 * STRIPE
HF = HID // 2

_mesh = plsc.VectorSubcoreMesh(core_axis_name="c", subcore_axis_name="s")
_sc_params = pltpu.CompilerParams(use_tc_tiling_on_sc=False,
                                  needs_layout_passes=False)


def _deg_body(ei3_hbm, deg_hbm, ones_v, zero_v, idx_v, spmem):
    c = lax.axis_index("c")
    s = lax.axis_index("s")

    def fill_ones(i, _):
        ones_v[i] = jnp.ones((16,), jnp.float32)
        return 0

    lax.fori_loop(0, CH, fill_ones, 0)

    def fill_zero(i, _):
        zero_v[i] = jnp.zeros((16,), jnp.float32)
        return 0

    lax.fori_loop(0, STRIPE, fill_zero, 0)

    pltpu.sync_copy(ei3_hbm.at[c, s], idx_v)

    @pl.when(s < NS - 1)
    def _():
        pltpu.sync_copy(zero_v, spmem.at[pl.ds(s * STRIPE, STRIPE)])

    @pl.when(s == NS - 1)
    def _():
        pltpu.sync_copy(zero_v.at[pl.ds(0, STRIPE_LAST)],
                        spmem.at[pl.ds(s * STRIPE, STRIPE_LAST)])

    plsc.subcore_barrier()

    def chunk(j, _):
        pltpu.sync_copy(ones_v, spmem.at[idx_v.at[j]], add=True)
        return 0

    lax.fori_loop(0, NCHUNK, chunk, 0)
    plsc.subcore_barrier()

    @pl.when(s < NS - 1)
    def _():
        pltpu.sync_copy(spmem.at[pl.ds(s * STRIPE, STRIPE)],
                        deg_hbm.at[c, pl.ds(s * STRIPE, STRIPE)])

    @pl.when(s == NS - 1)
    def _():
        pltpu.sync_copy(spmem.at[pl.ds(s * STRIPE, STRIPE_LAST)],
                        deg_hbm.at[c, pl.ds(s * STRIPE, STRIPE_LAST)])


_deg_kernel = pl.kernel(
    _deg_body,
    out_type=jax.ShapeDtypeStruct((2, N, 16), jnp.float32),
    mesh=_mesh,
    compiler_params=_sc_params,
    scratch_types=[
        pltpu.VMEM((CH, 16), jnp.float32),
        pltpu.VMEM((STRIPE, 16), jnp.float32),
        pltpu.VMEM((NCHUNK, CH), jnp.int32),
        pltpu.VMEM_SHARED((N, 16), jnp.float32),
    ],
)


def _msg_body(hs_hbm, src_hbm, ei3_hbm, ew_hbm, agg_hbm,
              src_v, dst_v, ew_v, rows_v, rows_w, spmem, sem, sem2):
    c = lax.axis_index("c")
    s = lax.axis_index("s")
    base = s * EPT

    pltpu.sync_copy(src_hbm.at[pl.ds(base, EPT)], src_v)
    pltpu.sync_copy(ei3_hbm.at[1, s], dst_v)
    pltpu.sync_copy(ew_hbm.at[pl.ds(base, EPT)], ew_v)

    def zr(i, _):
        for k in range(HF // 16):
            rows_v[i, pl.ds(k * 16, 16)] = jnp.zeros((16,), jnp.float32)
        return 0

    lax.fori_loop(0, 80, zr, 0)
    nz = lax.select(s < NS - 1, STRIPE // 80, STRIPE_LAST // 80)

    def zcopy(k, _):
        pltpu.sync_copy(rows_v.at[pl.ds(0, 80)],
                        spmem.at[pl.ds(s * STRIPE + k * 80, 80)])
        return 0

    lax.fori_loop(0, nz, zcopy, 0)
    plsc.subcore_barrier()

    def start_gather(j, buf, sm):
        idx = src_v.at[pl.ds(j * CH, CH)]
        pltpu.async_copy(hs_hbm.at[c].at[idx], buf, sm)

    def scale_scatter(j, buf):
        def row(r, _):
            for u in range(2):
                rr = r * 2 + u
                f = plsc.load_gather(
                    ew_v,
                    [jnp.broadcast_to(j * CH + rr, (16,)).astype(jnp.int32)])
                for k in range(HF // 16):
                    sl = pl.ds(k * 16, 16)
                    buf[rr, sl] = buf[rr, sl] * f
            return 0

        lax.fori_loop(0, CH // 2, row, 0)
        pltpu.sync_copy(buf, spmem.at[dst_v.at[j]], add=True)

    start_gather(0, rows_v, sem)

    def pair(k, _):
        j0 = 2 * k
        pltpu.make_async_copy(hs_hbm.at[c].at[src_v.at[pl.ds(0, CH)]],
                              rows_v, sem).wait()
        start_gather(j0 + 1, rows_w, sem2)
        scale_scatter(j0, rows_v)
        pltpu.make_async_copy(hs_hbm.at[c].at[src_v.at[pl.ds(0, CH)]],
                              rows_w, sem2).wait()
        start_gather(j0 + 2, rows_v, sem)
        scale_scatter(j0 + 1, rows_w)
        return 0

    lax.fori_loop(0, (NCHUNK - 1) // 2, pair, 0)
    pltpu.make_async_copy(hs_hbm.at[c].at[src_v.at[pl.ds(0, CH)]],
                          rows_v, sem).wait()
    scale_scatter(NCHUNK - 1, rows_v)
    plsc.subcore_barrier()

    @pl.when(s < NS - 1)
    def _():
        pltpu.sync_copy(spmem.at[pl.ds(s * STRIPE, STRIPE)],
                        agg_hbm.at[c, pl.ds(s * STRIPE, STRIPE)])

    @pl.when(s == NS - 1)
    def _():
        pltpu.sync_copy(spmem.at[pl.ds(s * STRIPE, STRIPE_LAST)],
                        agg_hbm.at[c, pl.ds(s * STRIPE, STRIPE_LAST)])


_msg_kernel = pl.kernel(
    _msg_body,
    out_type=jax.ShapeDtypeStruct((2, N, HF), jnp.float32),
    mesh=_mesh,
    compiler_params=_sc_params,
    scratch_types=[
        pltpu.VMEM((EPT,), jnp.int32),
        pltpu.VMEM((NCHUNK, CH), jnp.int32),
        pltpu.VMEM((EPT,), jnp.float32),
        pltpu.VMEM((CH, HF), jnp.float32),
        pltpu.VMEM((CH, HF), jnp.float32),
        pltpu.VMEM_SHARED((N, HF), jnp.float32),
        pltpu.SemaphoreType.DMA,
        pltpu.SemaphoreType.DMA,
    ],
)


def _norm_body(deg_ref, out_ref):
    d = deg_ref[0, :, :1]
    out_ref[0] = lax.rsqrt(jnp.maximum(d, 1.0))


def _norm_kernel(deg16):
    bn = 2000
    return pl.pallas_call(
        _norm_body,
        grid=(2, N // bn),
        in_specs=[pl.BlockSpec((1, bn, 16), lambda a, i: (a, i, 0))],
        out_specs=pl.BlockSpec((1, bn, 1), lambda a, i: (a, i, 0)),
        out_shape=jax.ShapeDtypeStruct((2, N, 1), jnp.float32),
    )(deg16)


def _mmA_body(x_ref, onorm_ref, w_ref, o_ref):
    o_ref[0] = jnp.dot(x_ref[...] * onorm_ref[...], w_ref[...],
                       preferred_element_type=jnp.float32)


def _mmA(x, onorm, W1):
    br = 1000
    return pl.pallas_call(
        _mmA_body,
        grid=(N // br, 2),
        in_specs=[
            pl.BlockSpec((br, F_IN), lambda i, j: (i, 0)),
            pl.BlockSpec((br, 1), lambda i, j: (i, 0)),
            pl.BlockSpec((F_IN, HF), lambda i, j: (0, j)),
        ],
        out_specs=pl.BlockSpec((1, br, HF), lambda i, j: (j, i, 0)),
        out_shape=jax.ShapeDtypeStruct((2, N, HF), jnp.float32),
    )(x, onorm, W1)


def _mmB_body(alo_ref, ahi_ref, inorm_ref, onorm_ref, b_ref, w_ref, o_ref):
    a = jnp.concatenate([alo_ref[0], ahi_ref[0]], axis=1)
    h = jnp.maximum(a * inorm_ref[...] + b_ref[...], 0.0) * onorm_ref[...]
    o_ref[0] = jnp.dot(h, w_ref[...], preferred_element_type=jnp.float32)


def _mmB(agg, inorm, onorm, b, W):
    br = 1000
    return pl.pallas_call(
        _mmB_body,
        grid=(N // br, 2),
        in_specs=[
            pl.BlockSpec((1, br, HF), lambda i, j: (0, i, 0)),
            pl.BlockSpec((1, br, HF), lambda i, j: (1, i, 0)),
            pl.BlockSpec((br, 1), lambda i, j: (i, 0)),
            pl.BlockSpec((br, 1), lambda i, j: (i, 0)),
            pl.BlockSpec((1, HID), lambda i, j: (0, 0)),
            pl.BlockSpec((HID, HF), lambda i, j: (0, j)),
        ],
        out_specs=pl.BlockSpec((1, br, HF), lambda i, j: (j, i, 0)),
        out_shape=jax.ShapeDtypeStruct((2, N, HF), jnp.float32),
    )(agg, agg, inorm, onorm, b, W)


def _mmC1_body(alo_ref, ahi_ref, inorm_ref, b_ref, we_ref, be_ref, o_ref):
    a = jnp.concatenate([alo_ref[0], ahi_ref[0]], axis=1)
    h = jnp.maximum(a * inorm_ref[...] + b_ref[...], 0.0)
    z = jnp.dot(h, we_ref[...], preferred_element_type=jnp.float32)
    o_ref[...] = jnp.maximum(z + be_ref[...], 0.0)


def _mmC1(agg, inorm, b, We, be):
    br = 1000
    return pl.pallas_call(
        _mmC1_body,
        grid=(N // br,),
        in_specs=[
            pl.BlockSpec((1, br, HF), lambda i: (0, i, 0)),
            pl.BlockSpec((1, br, HF), lambda i: (1, i, 0)),
            pl.BlockSpec((br, 1), lambda i: (i, 0)),
            pl.BlockSpec((1, HID), lambda i: (0, 0)),
            pl.BlockSpec((HID, H1), lambda i: (0, 0)),
            pl.BlockSpec((1, H1), lambda i: (0, 0)),
        ],
        out_specs=pl.BlockSpec((br, H1), lambda i: (i, 0)),
        out_shape=jax.ShapeDtypeStruct((N, H1), jnp.float32),
    )(agg, agg, inorm, b, We, be)


def _mmC2_body(zi_ref, zj_ref, o_ref):
    o_ref[...] = lax.dot_general(
        zi_ref[...], zj_ref[...], (((1,), (1,)), ((), ())),
        preferred_element_type=jnp.float32)


def _mmC2(z):
    bi = 400
    return pl.pallas_call(
        _mmC2_body,
        grid=(N // bi,),
        in_specs=[
            pl.BlockSpec((bi, H1), lambda i: (i, 0)),
            pl.BlockSpec((N, H1), lambda i: (0, 0)),
        ],
        out_specs=pl.BlockSpec((bi, N), lambda i: (i, 0)),
        out_shape=jax.ShapeDtypeStruct((N, N), jnp.float32),
    )(z, z)


def kernel(x, edge_index, edge_weight, W1, b1, W2, b2, We, be):
    ei3 = edge_index.reshape(2, NS, NCHUNK, CH)
    b1r = b1.reshape(1, HID)
    b2r = b2.reshape(1, HID)
    ber = be.reshape(1, H1)

    src_flat = edge_index[0]

    deg16 = _deg_kernel(ei3)
    norms = _norm_kernel(deg16)
    onorm = norms[0]
    inorm = norms[1]

    hs1 = _mmA(x, onorm, W1)
    agg1 = _msg_kernel(hs1, src_flat, ei3, edge_weight)
    hs2 = _mmB(agg1, inorm, onorm, b1r, W2)
    agg2 = _msg_kernel(hs2, src_flat, ei3, edge_weight)
    z = _mmC1(agg2, inorm, b2r, We, ber)
    adj = _mmC2(z)
    return (adj, z)
